# baseline (speedup 1.0000x reference)
import functools

import jax
import jax.numpy as jnp
from jax import lax
from jax.experimental import pallas as pl
from jax.experimental.pallas import tpu as pltpu
from jax.experimental.pallas import tpu_sc as plsc

NC = 2
NS = 16
L = 16
D = 16
CH = 128
B = 16384
NROWS = B // CH
NW = NC * NS
NCH = NROWS // NW


def _gmf_body(uidx_hbm, iidx_hbm, ut_hbm, it_hbm, wvec_hbm, bvec_hbm,
              out_hbm, uidx_v, iidx_v, urows_v, irows_v, out_v, wv_v, bv_v,
              sem_u, sem_i):
    c = lax.axis_index("c")
    s = lax.axis_index("s")
    wid = s * NC + c

    pltpu.sync_copy(wvec_hbm, wv_v)
    pltpu.sync_copy(bvec_hbm, bv_v)
    wv = wv_v[...]
    bv = bv_v[...]
    lanes = lax.iota(jnp.int32, L)

    for j in range(NCH):
        row = wid * NCH + j
        pltpu.sync_copy(uidx_hbm.at[row], uidx_v)
        pltpu.sync_copy(iidx_hbm.at[row], iidx_v)
        cu = pltpu.async_copy(ut_hbm.at[uidx_v], urows_v, sem_u)
        ci = pltpu.async_copy(it_hbm.at[iidx_v], irows_v, sem_i)
        cu.wait()
        ci.wait()

        for g in range(CH // L):
            rows = lanes + (g * L)

            def dbody(d, acc):
                dd = jnp.full((L,), d, dtype=jnp.int32)
                ucol = plsc.load_gather(urows_v, [rows, dd])
                icol = plsc.load_gather(irows_v, [rows, dd])
                return acc + ucol * icol

            dot = lax.fori_loop(0, D, dbody, jnp.zeros((L,), jnp.float32))
            logits = dot * wv + bv
            out_v[pl.ds(g * L, L)] = 1.0 / (1.0 + jnp.exp(-logits))

        pltpu.sync_copy(out_v, out_hbm.at[row])


@jax.jit
def _gmf(uidx2, iidx2, user_table, item_table, wvec, bvec):
    mesh = plsc.VectorSubcoreMesh(
        core_axis_name="c", subcore_axis_name="s",
        num_cores=NC, num_subcores=NS)
    run = functools.partial(
        pl.kernel,
        out_type=jax.ShapeDtypeStruct((NROWS, CH), jnp.float32),
        mesh=mesh,
        compiler_params=pltpu.CompilerParams(
            needs_layout_passes=False, use_tc_tiling_on_sc=False),
        scratch_types=[
            pltpu.VMEM((CH,), jnp.int32),
            pltpu.VMEM((CH,), jnp.int32),
            pltpu.VMEM((CH, D), jnp.float32),
            pltpu.VMEM((CH, D), jnp.float32),
            pltpu.VMEM((CH,), jnp.float32),
            pltpu.VMEM((L,), jnp.float32),
            pltpu.VMEM((L,), jnp.float32),
            pltpu.SemaphoreType.DMA,
            pltpu.SemaphoreType.DMA,
        ],
    )(_gmf_body)
    return run(uidx2, iidx2, user_table, item_table, wvec, bvec)


def kernel(user_indices, item_indices, user_table, item_table, dense_w, dense_b):
    uidx2 = user_indices.astype(jnp.int32).reshape(NROWS, CH)
    iidx2 = item_indices.astype(jnp.int32).reshape(NROWS, CH)
    wvec = jnp.full((L,), dense_w[0, 0], dtype=jnp.float32)
    bvec = jnp.full((L,), dense_b[0], dtype=jnp.float32)
    out = _gmf(uidx2, iidx2, user_table, item_table, wvec, bvec)
    return out.reshape(B, 1)
